# baseline (speedup 1.0000x reference)
import functools

import jax
import jax.numpy as jnp
from jax import lax
from jax.experimental import pallas as pl
from jax.experimental.pallas import tpu as pltpu
from jax.experimental.pallas import tpu_sc as plsc

B = 16
L = 16
NC, NS = 2, 16
NW = NC * NS


def _lane(vec, s, iota):
    return jnp.sum(jnp.where(iota == s, vec, 0))


def _seg_bounds(bidx_v, r, iota):
    lo = jnp.zeros((L,), jnp.int32)
    hi = jnp.full((L,), r, jnp.int32)
    steps = max(1, (r).bit_length())

    def body(_, carry):
        lo, hi = carry
        active = lo < hi
        mid = lax.div(lo + hi, 2)
        vals = plsc.load_gather(bidx_v, [jnp.minimum(mid, r - 1)])
        pred = vals < iota
        lo = jnp.where(active & pred, mid + 1, lo)
        hi = jnp.where(active & jnp.logical_not(pred), mid, hi)
        return lo, hi

    lo, hi = lax.fori_loop(0, steps, body, (lo, hi))
    starts = [_lane(lo, s, iota) for s in range(B)]
    ends = starts[1:] + [jnp.int32(r)]
    return starts, ends


def _make_sums_kernel(n, c, r, ch):
    nch = r // ch
    mesh = plsc.VectorSubcoreMesh(core_axis_name="c", subcore_axis_name="s")

    @functools.partial(
        pl.kernel,
        out_type=[
            jax.ShapeDtypeStruct((NW, B * c), jnp.float32),
            jax.ShapeDtypeStruct((NW * L,), jnp.float32),
        ],
        mesh=mesh,
        compiler_params=pltpu.CompilerParams(needs_layout_passes=False),
        scratch_types=[
            pltpu.VMEM((r,), jnp.int32),
            pltpu.VMEM((ch * c,), jnp.float32),
            pltpu.VMEM((ch * c,), jnp.float32),
            pltpu.VMEM((B * c,), jnp.float32),
            pltpu.VMEM((L,), jnp.float32),
            pltpu.SemaphoreType.DMA,
            pltpu.SemaphoreType.DMA,
            pltpu.SemaphoreType.DMA,
        ],
    )
    def sums_kernel(feat_hbm, bidx_hbm, psum_hbm, pcnt_hbm,
                    bidx_v, f0, f1, acc_v, cnt_v, semi, sem0, sem1):
        wid = lax.axis_index("c") * NS + lax.axis_index("s")
        base = wid * r
        iota = lax.iota(jnp.int32, L)
        zero16 = jnp.zeros((L,), jnp.float32)

        pltpu.async_copy(bidx_hbm.at[pl.ds(base, r)], bidx_v, semi)
        pltpu.async_copy(feat_hbm.at[pl.ds(base * c, ch * c)], f0, sem0)
        pltpu.async_copy(feat_hbm.at[pl.ds((base + ch) * c, ch * c)], f1, sem1)

        for q in range(B * c // L):
            acc_v[pl.ds(q * L, L)] = zero16

        pltpu.make_async_copy(bidx_hbm.at[pl.ds(base, r)], bidx_v, semi).wait()
        starts, ends = _seg_bounds(bidx_v, r, iota)

        cnt_f = zero16
        for s in range(B):
            cnt_f = jnp.where(iota == s,
                              (ends[s] - starts[s]).astype(jnp.float32), cnt_f)
        cnt_v[...] = cnt_f

        def chunk_pair(p, carry):
            for b in range(2):
                j = p * 2 + b
                cb = j * ch
                fb = f0 if b == 0 else f1
                semb = sem0 if b == 0 else sem1
                pltpu.make_async_copy(
                    feat_hbm.at[pl.ds((base + cb) * c, ch * c)], fb, semb
                ).wait()
                for s in range(B):
                    lo = jnp.maximum(starts[s] - cb, 0)
                    hi = jnp.minimum(ends[s] - cb, ch)

                    @pl.when(hi > lo)
                    def _(s=s, lo=lo, hi=hi, fb=fb):
                        acc8 = tuple(
                            acc_v[pl.ds(s * c + k * L, L)] for k in range(c // L)
                        )

                        def row_body(rr, a8):
                            return tuple(
                                a + fb[pl.ds(rr * c + k * L, L)]
                                for k, a in enumerate(a8)
                            )

                        acc8 = lax.fori_loop(lo, hi, row_body, acc8)
                        for k in range(c // L):
                            acc_v[pl.ds(s * c + k * L, L)] = acc8[k]

                nxt = j + 2

                @pl.when(nxt < nch)
                def _(nxt=nxt, fb=fb, semb=semb):
                    pltpu.async_copy(
                        feat_hbm.at[pl.ds((base + nxt * ch) * c, ch * c)],
                        fb, semb)
            return carry

        lax.fori_loop(0, nch // 2, chunk_pair, 0)

        pltpu.sync_copy(acc_v, psum_hbm.at[wid])
        pltpu.sync_copy(cnt_v, pcnt_hbm.at[pl.ds(wid * L, L)])

    return sums_kernel


def _make_apply_kernel(n, c, r, ch):
    nch = r // ch
    mesh = plsc.VectorSubcoreMesh(core_axis_name="c", subcore_axis_name="s")

    @functools.partial(
        pl.kernel,
        out_type=jax.ShapeDtypeStruct((n * c,), jnp.float32),
        mesh=mesh,
        compiler_params=pltpu.CompilerParams(needs_layout_passes=False),
        scratch_types=[
            pltpu.VMEM((r,), jnp.int32),
            pltpu.VMEM((ch * c,), jnp.float32),
            pltpu.VMEM((ch * c,), jnp.float32),
            pltpu.VMEM((ch * c,), jnp.float32),
            pltpu.VMEM((ch * c,), jnp.float32),
            pltpu.VMEM((B * c,), jnp.float32),
            pltpu.VMEM((B * c,), jnp.float32),
            pltpu.VMEM((B * c,), jnp.float32),
            pltpu.VMEM((c * L,), jnp.float32),
            pltpu.VMEM((B * c,), jnp.float32),
            pltpu.VMEM((NW * L,), jnp.float32),
            pltpu.VMEM((4 * L,), jnp.float32),
            pltpu.SemaphoreType.DMA,
            pltpu.SemaphoreType.DMA,
            pltpu.SemaphoreType.DMA,
            pltpu.SemaphoreType.DMA,
            pltpu.SemaphoreType.DMA,
            pltpu.SemaphoreType.DMA,
            pltpu.SemaphoreType.DMA,
        ],
    )
    def apply_kernel(feat_hbm, bidx_hbm, psum_hbm, pcnt_hbm, wvec_hbm, out_hbm,
                     bidx_v, f0, f1, o0, o1, pb0, pb1, tot_v, mt_v, gates_v,
                     pcv, wv, semi, sem0, sem1, semo0, semo1, semp0, semp1):
        wid = lax.axis_index("c") * NS + lax.axis_index("s")
        base = wid * r
        iota = lax.iota(jnp.int32, L)
        zero16 = jnp.zeros((L,), jnp.float32)

        pltpu.async_copy(bidx_hbm.at[pl.ds(base, r)], bidx_v, semi)
        pltpu.async_copy(feat_hbm.at[pl.ds(base * c, ch * c)], f0, sem0)
        pltpu.async_copy(feat_hbm.at[pl.ds((base + ch) * c, ch * c)], f1, sem1)
        pltpu.async_copy(psum_hbm.at[0], pb0, semp0)
        pltpu.async_copy(psum_hbm.at[1], pb1, semp1)
        pltpu.sync_copy(pcnt_hbm, pcv)
        pltpu.sync_copy(wvec_hbm, wv)

        for q in range(B * c // L):
            tot_v[pl.ds(q * L, L)] = zero16

        def comb_pair(p, carry):
            for b in range(2):
                w = p * 2 + b
                pbb = pb0 if b == 0 else pb1
                semb = semp0 if b == 0 else semp1
                pltpu.make_async_copy(psum_hbm.at[w], pbb, semb).wait()

                def add_body(q, carry2, pbb=pbb):
                    tot_v[pl.ds(q * L, L)] = (tot_v[pl.ds(q * L, L)]
                                              + pbb[pl.ds(q * L, L)])
                    return carry2

                lax.fori_loop(0, B * c // L, add_body, 0)
                nxt = w + 2

                @pl.when(nxt < NW)
                def _(nxt=nxt, pbb=pbb, semb=semb):
                    pltpu.async_copy(psum_hbm.at[nxt], pbb, semb)
            return carry

        lax.fori_loop(0, NW // 2, comb_pair, 0)

        ctot = zero16
        for w in range(NW):
            ctot = ctot + pcv[pl.ds(w * L, L)]
        recip = 1.0 / jnp.maximum(ctot, 1.0)

        idx0 = iota * c

        def tr_body(cc, carry):
            m = plsc.load_gather(tot_v, [idx0 + cc]) * recip
            mt_v[pl.ds(cc * L, L)] = m
            return carry

        lax.fori_loop(0, c, tr_body, 0)

        w0 = wv[pl.ds(0, L)]
        w1 = wv[pl.ds(L, L)]
        w2 = wv[pl.ds(2 * L, L)]

        def conv_body(cc, carry):
            mid = mt_v[pl.ds(cc * L, L)]
            left = jnp.where(cc > 0,
                             mt_v[pl.ds(jnp.maximum(cc - 1, 0) * L, L)], zero16)
            right = jnp.where(cc < c - 1,
                              mt_v[pl.ds(jnp.minimum(cc + 1, c - 1) * L, L)],
                              zero16)
            y = w0 * left + w1 * mid + w2 * right
            g = 1.0 / (1.0 + jnp.exp(-y))
            plsc.store_scatter(gates_v, [idx0 + cc], g)
            return carry

        lax.fori_loop(0, c, conv_body, 0)

        pltpu.make_async_copy(bidx_hbm.at[pl.ds(base, r)], bidx_v, semi).wait()
        starts, ends = _seg_bounds(bidx_v, r, iota)

        def chunk_pair(p, carry):
            for b in range(2):
                j = p * 2 + b
                cb = j * ch
                fb = f0 if b == 0 else f1
                ob = o0 if b == 0 else o1
                semb = sem0 if b == 0 else sem1
                semob = semo0 if b == 0 else semo1
                pltpu.make_async_copy(
                    feat_hbm.at[pl.ds((base + cb) * c, ch * c)], fb, semb
                ).wait()

                @pl.when(j >= 2)
                def _(j=j, ob=ob, semob=semob):
                    pltpu.make_async_copy(
                        ob,
                        out_hbm.at[pl.ds((base + (j - 2) * ch) * c, ch * c)],
                        semob).wait()

                for s in range(B):
                    lo = jnp.maximum(starts[s] - cb, 0)
                    hi = jnp.minimum(ends[s] - cb, ch)

                    @pl.when(hi > lo)
                    def _(s=s, lo=lo, hi=hi, fb=fb, ob=ob):
                        g8 = tuple(
                            gates_v[pl.ds(s * c + k * L, L)]
                            for k in range(c // L)
                        )

                        def row_body(rr, carry2):
                            for k in range(c // L):
                                ob[pl.ds(rr * c + k * L, L)] = (
                                    fb[pl.ds(rr * c + k * L, L)] * g8[k])
                            return carry2

                        lax.fori_loop(lo, hi, row_body, 0)

                pltpu.async_copy(
                    ob, out_hbm.at[pl.ds((base + cb) * c, ch * c)], semob)
                nxt = j + 2

                @pl.when(nxt < nch)
                def _(nxt=nxt, fb=fb, semb=semb):
                    pltpu.async_copy(
                        feat_hbm.at[pl.ds((base + nxt * ch) * c, ch * c)],
                        fb, semb)
            return carry

        lax.fori_loop(0, nch // 2, chunk_pair, 0)

        for j in (nch - 2, nch - 1):
            semob = semo0 if j % 2 == 0 else semo1
            pltpu.make_async_copy(
                o0 if j % 2 == 0 else o1,
                out_hbm.at[pl.ds((base + j * ch) * c, ch * c)], semob).wait()

    return apply_kernel


def kernel(features, batch_idx, W):
    n, c = features.shape
    assert n % NW == 0 and c % L == 0
    r = n // NW
    ch_a = 250 if r % 250 == 0 else 200
    ch_b = 200 if r % 200 == 0 else 100
    assert r % ch_a == 0 and r % ch_b == 0 and (r // ch_a) % 2 == 0 \
        and (r // ch_b) % 2 == 0

    featflat = features.reshape(-1)
    wflat = W.reshape(3)
    wvec = jnp.concatenate([
        jnp.full((L,), wflat[0], jnp.float32),
        jnp.full((L,), wflat[1], jnp.float32),
        jnp.full((L,), wflat[2], jnp.float32),
        jnp.zeros((L,), jnp.float32),
    ])

    psum, pcnt = _make_sums_kernel(n, c, r, ch_a)(featflat, batch_idx)
    outflat = _make_apply_kernel(n, c, r, ch_b)(
        featflat, batch_idx, psum, pcnt, wvec)
    return outflat.reshape(n, c)


# speedup vs baseline: 1.7906x; 1.7906x over previous
import functools

import jax
import jax.numpy as jnp
from jax import lax
from jax.experimental import pallas as pl
from jax.experimental.pallas import tpu as pltpu
from jax.experimental.pallas import tpu_sc as plsc

B = 16
L = 16
NC, NS = 2, 16
NW = NC * NS


def _lane(vec, s, iota):
    return jnp.sum(jnp.where(iota == s, vec, 0))


def _seg_bounds(bidx_v, r, iota):
    lo = jnp.zeros((L,), jnp.int32)
    hi = jnp.full((L,), r, jnp.int32)
    steps = max(1, (r).bit_length())

    def body(_, carry):
        lo, hi = carry
        active = lo < hi
        mid = lax.div(lo + hi, 2)
        vals = plsc.load_gather(bidx_v, [jnp.minimum(mid, r - 1)])
        pred = vals < iota
        lo = jnp.where(active & pred, mid + 1, lo)
        hi = jnp.where(active & jnp.logical_not(pred), mid, hi)
        return lo, hi

    lo, hi = lax.fori_loop(0, steps, body, (lo, hi))
    starts = [_lane(lo, s, iota) for s in range(B)]
    ends = starts[1:] + [jnp.int32(r)]
    return starts, ends


def _make_sums_kernel(n, c, r, ch):
    nch = r // ch
    mesh = plsc.VectorSubcoreMesh(core_axis_name="c", subcore_axis_name="s")

    @functools.partial(
        pl.kernel,
        out_type=[
            jax.ShapeDtypeStruct((NW, B * c), jnp.float32),
            jax.ShapeDtypeStruct((NW * L,), jnp.float32),
        ],
        mesh=mesh,
        compiler_params=pltpu.CompilerParams(needs_layout_passes=False),
        scratch_types=[
            pltpu.VMEM((r,), jnp.int32),
            pltpu.VMEM((ch * c,), jnp.float32),
            pltpu.VMEM((ch * c,), jnp.float32),
            pltpu.VMEM((B * c,), jnp.float32),
            pltpu.VMEM((L,), jnp.float32),
            pltpu.SemaphoreType.DMA,
            pltpu.SemaphoreType.DMA,
            pltpu.SemaphoreType.DMA,
        ],
    )
    def sums_kernel(feat_hbm, bidx_hbm, psum_hbm, pcnt_hbm,
                    bidx_v, f0, f1, acc_v, cnt_v, semi, sem0, sem1):
        wid = lax.axis_index("c") * NS + lax.axis_index("s")
        base = wid * r
        iota = lax.iota(jnp.int32, L)
        zero16 = jnp.zeros((L,), jnp.float32)

        pltpu.async_copy(bidx_hbm.at[pl.ds(base, r)], bidx_v, semi)
        pltpu.async_copy(feat_hbm.at[pl.ds(base * c, ch * c)], f0, sem0)
        pltpu.async_copy(feat_hbm.at[pl.ds((base + ch) * c, ch * c)], f1, sem1)

        for q in range(B * c // L):
            acc_v[pl.ds(q * L, L)] = zero16

        pltpu.make_async_copy(bidx_hbm.at[pl.ds(base, r)], bidx_v, semi).wait()
        starts, ends = _seg_bounds(bidx_v, r, iota)

        cnt_f = zero16
        for s in range(B):
            cnt_f = jnp.where(iota == s,
                              (ends[s] - starts[s]).astype(jnp.float32), cnt_f)
        cnt_v[...] = cnt_f

        def chunk_pair(p, carry):
            for b in range(2):
                j = p * 2 + b
                cb = j * ch
                fb = f0 if b == 0 else f1
                semb = sem0 if b == 0 else sem1
                pltpu.make_async_copy(
                    feat_hbm.at[pl.ds((base + cb) * c, ch * c)], fb, semb
                ).wait()
                for s in range(B):
                    lo = jnp.maximum(starts[s] - cb, 0)
                    hi = jnp.minimum(ends[s] - cb, ch)

                    @pl.when(hi > lo)
                    def _(s=s, lo=lo, hi=hi, fb=fb):
                        acc8 = tuple(
                            acc_v[pl.ds(s * c + k * L, L)] for k in range(c // L)
                        )

                        def row_body(rr, a8):
                            return tuple(
                                a + fb[pl.ds(rr * c + k * L, L)]
                                for k, a in enumerate(a8)
                            )

                        acc8 = lax.fori_loop(lo, hi, row_body, acc8)
                        for k in range(c // L):
                            acc_v[pl.ds(s * c + k * L, L)] = acc8[k]

                nxt = j + 2

                @pl.when(nxt < nch)
                def _(nxt=nxt, fb=fb, semb=semb):
                    pltpu.async_copy(
                        feat_hbm.at[pl.ds((base + nxt * ch) * c, ch * c)],
                        fb, semb)
            return carry

        lax.fori_loop(0, nch // 2, chunk_pair, 0)

        pltpu.sync_copy(acc_v, psum_hbm.at[wid])
        pltpu.sync_copy(cnt_v, pcnt_hbm.at[pl.ds(wid * L, L)])

    return sums_kernel


def _make_apply_kernel(n, c, r, ch):
    nch = r // ch
    mesh = plsc.VectorSubcoreMesh(core_axis_name="c", subcore_axis_name="s")

    @functools.partial(
        pl.kernel,
        out_type=jax.ShapeDtypeStruct((n * c,), jnp.float32),
        mesh=mesh,
        compiler_params=pltpu.CompilerParams(needs_layout_passes=False),
        scratch_types=[
            pltpu.VMEM((r,), jnp.int32),
            pltpu.VMEM((ch * c,), jnp.float32),
            pltpu.VMEM((ch * c,), jnp.float32),
            pltpu.VMEM((ch * c,), jnp.float32),
            pltpu.VMEM((ch * c,), jnp.float32),
            pltpu.VMEM((B * c,), jnp.float32),
            pltpu.VMEM((B * c,), jnp.float32),
            pltpu.VMEM((B * c,), jnp.float32),
            pltpu.VMEM((c * L,), jnp.float32),
            pltpu.VMEM((B * c,), jnp.float32),
            pltpu.VMEM((NW * L,), jnp.float32),
            pltpu.VMEM((4 * L,), jnp.float32),
            pltpu.SemaphoreType.DMA,
            pltpu.SemaphoreType.DMA,
            pltpu.SemaphoreType.DMA,
            pltpu.SemaphoreType.DMA,
            pltpu.SemaphoreType.DMA,
            pltpu.SemaphoreType.DMA,
            pltpu.SemaphoreType.DMA,
        ],
    )
    def apply_kernel(feat_hbm, bidx_hbm, psum_hbm, pcnt_hbm, wvec_hbm, out_hbm,
                     bidx_v, f0, f1, o0, o1, pb0, pb1, tot_v, mt_v, gates_v,
                     pcv, wv, semi, sem0, sem1, semo0, semo1, semp0, semp1):
        wid = lax.axis_index("c") * NS + lax.axis_index("s")
        base = wid * r
        iota = lax.iota(jnp.int32, L)
        zero16 = jnp.zeros((L,), jnp.float32)

        pltpu.async_copy(bidx_hbm.at[pl.ds(base, r)], bidx_v, semi)
        pltpu.async_copy(feat_hbm.at[pl.ds(base * c, ch * c)], f0, sem0)
        pltpu.async_copy(feat_hbm.at[pl.ds((base + ch) * c, ch * c)], f1, sem1)
        pltpu.async_copy(psum_hbm.at[0], pb0, semp0)
        pltpu.async_copy(psum_hbm.at[1], pb1, semp1)
        pltpu.sync_copy(pcnt_hbm, pcv)
        pltpu.sync_copy(wvec_hbm, wv)

        for q in range(B * c // L):
            tot_v[pl.ds(q * L, L)] = zero16

        def comb_pair(p, carry):
            for b in range(2):
                w = p * 2 + b
                pbb = pb0 if b == 0 else pb1
                semb = semp0 if b == 0 else semp1
                pltpu.make_async_copy(psum_hbm.at[w], pbb, semb).wait()

                @plsc.parallel_loop(0, B * c // L)
                def _(q, pbb=pbb):
                    tot_v[pl.ds(q * L, L)] = (tot_v[pl.ds(q * L, L)]
                                              + pbb[pl.ds(q * L, L)])
                nxt = w + 2

                @pl.when(nxt < NW)
                def _(nxt=nxt, pbb=pbb, semb=semb):
                    pltpu.async_copy(psum_hbm.at[nxt], pbb, semb)
            return carry

        lax.fori_loop(0, NW // 2, comb_pair, 0)

        ctot = zero16
        for w in range(NW):
            ctot = ctot + pcv[pl.ds(w * L, L)]
        recip = 1.0 / jnp.maximum(ctot, 1.0)

        idx0 = iota * c

        def tr_body(cc, carry):
            m = plsc.load_gather(tot_v, [idx0 + cc]) * recip
            mt_v[pl.ds(cc * L, L)] = m
            return carry

        lax.fori_loop(0, c, tr_body, 0)

        w0 = wv[pl.ds(0, L)]
        w1 = wv[pl.ds(L, L)]
        w2 = wv[pl.ds(2 * L, L)]

        def conv_body(cc, carry):
            mid = mt_v[pl.ds(cc * L, L)]
            left = jnp.where(cc > 0,
                             mt_v[pl.ds(jnp.maximum(cc - 1, 0) * L, L)], zero16)
            right = jnp.where(cc < c - 1,
                              mt_v[pl.ds(jnp.minimum(cc + 1, c - 1) * L, L)],
                              zero16)
            y = w0 * left + w1 * mid + w2 * right
            g = 1.0 / (1.0 + jnp.exp(-y))
            plsc.store_scatter(gates_v, [idx0 + cc], g)
            return carry

        lax.fori_loop(0, c, conv_body, 0)

        pltpu.make_async_copy(bidx_hbm.at[pl.ds(base, r)], bidx_v, semi).wait()
        starts, ends = _seg_bounds(bidx_v, r, iota)

        def chunk_pair(p, carry):
            for b in range(2):
                j = p * 2 + b
                cb = j * ch
                fb = f0 if b == 0 else f1
                ob = o0 if b == 0 else o1
                semb = sem0 if b == 0 else sem1
                semob = semo0 if b == 0 else semo1
                pltpu.make_async_copy(
                    feat_hbm.at[pl.ds((base + cb) * c, ch * c)], fb, semb
                ).wait()

                @pl.when(j >= 2)
                def _(j=j, ob=ob, semob=semob):
                    pltpu.make_async_copy(
                        ob,
                        out_hbm.at[pl.ds((base + (j - 2) * ch) * c, ch * c)],
                        semob).wait()

                for s in range(B):
                    lo = jnp.maximum(starts[s] - cb, 0)
                    hi = jnp.minimum(ends[s] - cb, ch)

                    @pl.when(hi > lo)
                    def _(s=s, lo=lo, hi=hi, fb=fb, ob=ob):
                        g8 = tuple(
                            gates_v[pl.ds(s * c + k * L, L)]
                            for k in range(c // L)
                        )

                        @plsc.parallel_loop(lo, hi, unroll=2)
                        def _(rr, g8=g8, fb=fb, ob=ob):
                            for k in range(c // L):
                                ob[pl.ds(rr * c + k * L, L)] = (
                                    fb[pl.ds(rr * c + k * L, L)] * g8[k])

                pltpu.async_copy(
                    ob, out_hbm.at[pl.ds((base + cb) * c, ch * c)], semob)
                nxt = j + 2

                @pl.when(nxt < nch)
                def _(nxt=nxt, fb=fb, semb=semb):
                    pltpu.async_copy(
                        feat_hbm.at[pl.ds((base + nxt * ch) * c, ch * c)],
                        fb, semb)
            return carry

        lax.fori_loop(0, nch // 2, chunk_pair, 0)

        for j in (nch - 2, nch - 1):
            semob = semo0 if j % 2 == 0 else semo1
            pltpu.make_async_copy(
                o0 if j % 2 == 0 else o1,
                out_hbm.at[pl.ds((base + j * ch) * c, ch * c)], semob).wait()

    return apply_kernel


def kernel(features, batch_idx, W):
    n, c = features.shape
    assert n % NW == 0 and c % L == 0
    r = n // NW
    ch_a = 250 if r % 250 == 0 else 200
    ch_b = 200 if r % 200 == 0 else 100
    assert r % ch_a == 0 and r % ch_b == 0 and (r // ch_a) % 2 == 0 \
        and (r // ch_b) % 2 == 0

    featflat = features.reshape(-1)
    wflat = W.reshape(3)
    wvec = jnp.concatenate([
        jnp.full((L,), wflat[0], jnp.float32),
        jnp.full((L,), wflat[1], jnp.float32),
        jnp.full((L,), wflat[2], jnp.float32),
        jnp.zeros((L,), jnp.float32),
    ])

    psum, pcnt = _make_sums_kernel(n, c, r, ch_a)(featflat, batch_idx)
    outflat = _make_apply_kernel(n, c, r, ch_b)(
        featflat, batch_idx, psum, pcnt, wvec)
    return outflat.reshape(n, c)


# speedup vs baseline: 2.3627x; 1.3195x over previous
import functools

import jax
import jax.numpy as jnp
from jax import lax
from jax.experimental import pallas as pl
from jax.experimental.pallas import tpu as pltpu
from jax.experimental.pallas import tpu_sc as plsc

B = 16
L = 16
NC, NS = 2, 16
NW = NC * NS


def _lane(vec, s, iota):
    return jnp.sum(jnp.where(iota == s, vec, 0))


def _seg_bounds(bidx_v, r, iota):
    lo = jnp.zeros((L,), jnp.int32)
    hi = jnp.full((L,), r, jnp.int32)
    steps = max(1, (r).bit_length())

    def body(_, carry):
        lo, hi = carry
        active = lo < hi
        mid = lax.div(lo + hi, 2)
        vals = plsc.load_gather(bidx_v, [jnp.minimum(mid, r - 1)])
        pred = vals < iota
        lo = jnp.where(active & pred, mid + 1, lo)
        hi = jnp.where(active & jnp.logical_not(pred), mid, hi)
        return lo, hi

    lo, hi = lax.fori_loop(0, steps, body, (lo, hi))
    starts = [_lane(lo, s, iota) for s in range(B)]
    ends = starts[1:] + [jnp.int32(r)]
    return starts, ends


def _make_sums_kernel(c, r, ch):
    nch = r // ch
    mesh = plsc.VectorSubcoreMesh(core_axis_name="c", subcore_axis_name="s")

    @functools.partial(
        pl.kernel,
        out_type=[
            jax.ShapeDtypeStruct((NW, B * c), jnp.float32),
            jax.ShapeDtypeStruct((NW * L,), jnp.float32),
        ],
        mesh=mesh,
        compiler_params=pltpu.CompilerParams(needs_layout_passes=False),
        scratch_types=[
            pltpu.VMEM((r,), jnp.int32),
            pltpu.VMEM((ch * c,), jnp.float32),
            pltpu.VMEM((ch * c,), jnp.float32),
            pltpu.VMEM((B * c,), jnp.float32),
            pltpu.VMEM((L,), jnp.float32),
            pltpu.SemaphoreType.DMA,
            pltpu.SemaphoreType.DMA,
            pltpu.SemaphoreType.DMA,
        ],
    )
    def sums_kernel(feat_hbm, bidx_hbm, psum_hbm, pcnt_hbm,
                    bidx_v, f0, f1, acc_v, cnt_v, semi, sem0, sem1):
        wid = lax.axis_index("c") * NS + lax.axis_index("s")
        base = wid * r
        iota = lax.iota(jnp.int32, L)
        zero16 = jnp.zeros((L,), jnp.float32)

        pltpu.async_copy(bidx_hbm.at[pl.ds(base, r)], bidx_v, semi)
        pltpu.async_copy(feat_hbm.at[pl.ds(base * c, ch * c)], f0, sem0)
        pltpu.async_copy(feat_hbm.at[pl.ds((base + ch) * c, ch * c)], f1, sem1)

        for q in range(B * c // L):
            acc_v[pl.ds(q * L, L)] = zero16

        pltpu.make_async_copy(bidx_hbm.at[pl.ds(base, r)], bidx_v, semi).wait()
        starts, ends = _seg_bounds(bidx_v, r, iota)

        cnt_f = zero16
        for s in range(B):
            cnt_f = jnp.where(iota == s,
                              (ends[s] - starts[s]).astype(jnp.float32), cnt_f)
        cnt_v[...] = cnt_f

        def chunk_pair(p, carry):
            for b in range(2):
                j = p * 2 + b
                cb = j * ch
                fb = f0 if b == 0 else f1
                semb = sem0 if b == 0 else sem1
                pltpu.make_async_copy(
                    feat_hbm.at[pl.ds((base + cb) * c, ch * c)], fb, semb
                ).wait()
                for s in range(B):
                    lo = jnp.maximum(starts[s] - cb, 0)
                    hi = jnp.minimum(ends[s] - cb, ch)

                    @pl.when(hi > lo)
                    def _(s=s, lo=lo, hi=hi, fb=fb):
                        acc8 = tuple(
                            acc_v[pl.ds(s * c + k * L, L)] for k in range(c // L)
                        )

                        @plsc.parallel_loop(lo, hi, unroll=2, carry=acc8)
                        def a8(rr, a8, fb=fb):
                            return tuple(
                                a + fb[pl.ds(rr * c + k * L, L)]
                                for k, a in enumerate(a8)
                            )

                        for k in range(c // L):
                            acc_v[pl.ds(s * c + k * L, L)] = a8[k]

                nxt = j + 2

                @pl.when(nxt < nch)
                def _(nxt=nxt, fb=fb, semb=semb):
                    pltpu.async_copy(
                        feat_hbm.at[pl.ds((base + nxt * ch) * c, ch * c)],
                        fb, semb)
            return carry

        lax.fori_loop(0, nch // 2, chunk_pair, 0)

        pltpu.sync_copy(acc_v, psum_hbm.at[wid])
        pltpu.sync_copy(cnt_v, pcnt_hbm.at[pl.ds(wid * L, L)])

    return sums_kernel


def _p1(bidx_ref, feat_ref, sums_ref, cnt_ref):
    i = pl.program_id(0)
    rb = feat_ref.shape[0]
    b = bidx_ref[0, 0, :]
    onehot = (b[:, None] == jax.lax.broadcasted_iota(jnp.int32, (rb, B), 1)
              ).astype(jnp.float32)
    part = jax.lax.dot_general(onehot, feat_ref[...],
                               (((0,), (0,)), ((), ())),
                               preferred_element_type=jnp.float32)
    pcnt = jnp.sum(onehot, axis=0)[:, None]

    @pl.when(i == 0)
    def _():
        sums_ref[...] = jnp.zeros_like(sums_ref)
        cnt_ref[...] = jnp.zeros_like(cnt_ref)

    sums_ref[...] += part
    cnt_ref[...] += jnp.broadcast_to(pcnt, cnt_ref.shape)


def _p2(tsums_ref, tcnt_ref, psum_ref, pcnt_ref, w_ref, gates_ref):
    sums = tsums_ref[...] + jnp.sum(psum_ref[...], axis=0)
    cnt = tcnt_ref[...] + jnp.sum(pcnt_ref[...], axis=0)[:, None]
    m = sums / jnp.maximum(cnt, 1.0)
    w0 = w_ref[0, 0]
    w1 = w_ref[0, 1]
    w2 = w_ref[0, 2]
    zero = jnp.zeros((m.shape[0], 1), jnp.float32)
    left = jnp.concatenate([zero, m[:, :-1]], axis=1)
    right = jnp.concatenate([m[:, 1:], zero], axis=1)
    y = w0 * left + w1 * m + w2 * right
    gates_ref[...] = jax.nn.sigmoid(y)


def _p3(bidx_ref, feat_ref, gates_ref, out_ref):
    rb = feat_ref.shape[0]
    b = bidx_ref[0, 0, :]
    onehot = (b[:, None] == jax.lax.broadcasted_iota(jnp.int32, (rb, B), 1)
              ).astype(jnp.float32)
    g = jax.lax.dot_general(onehot, gates_ref[...],
                            (((1,), (0,)), ((), ())),
                            preferred_element_type=jnp.float32)
    out_ref[...] = feat_ref[...] * g


N_SC_FRAC_NUM, N_SC_FRAC_DEN = 1, 2


def kernel(features, batch_idx, W):
    n, c = features.shape
    assert c % L == 0
    rb = 4000
    n_sc = (n * N_SC_FRAC_NUM // N_SC_FRAC_DEN) // (NW * rb) * (NW * rb) \
        // rb * rb
    n_sc = max(n_sc, NW * 500)
    r = n_sc // NW
    ch_a = 250
    assert r % ch_a == 0 and (r // ch_a) % 2 == 0, (r, ch_a)
    n_tc = n - n_sc

    featflat = features.reshape(-1)

    psum, pcnt = _make_sums_kernel(c, r, ch_a)(featflat, batch_idx)

    assert n_tc % rb == 0 and n_sc % rb == 0
    nb_tc = n_tc // rb
    blk0 = n_sc // rb
    bidx3_tc = batch_idx[n_sc:].reshape(nb_tc, 1, rb)
    tsums, tcnt = pl.pallas_call(
        _p1,
        grid=(nb_tc,),
        in_specs=[
            pl.BlockSpec((1, 1, rb), lambda i: (i, 0, 0)),
            pl.BlockSpec((rb, c), lambda i: (i + blk0, 0)),
        ],
        out_specs=[
            pl.BlockSpec((B, c), lambda i: (0, 0)),
            pl.BlockSpec((B, c), lambda i: (0, 0)),
        ],
        out_shape=[
            jax.ShapeDtypeStruct((B, c), jnp.float32),
            jax.ShapeDtypeStruct((B, c), jnp.float32),
        ],
    )(bidx3_tc, features)

    gates = pl.pallas_call(
        _p2,
        out_shape=jax.ShapeDtypeStruct((B, c), jnp.float32),
    )(tsums, tcnt, psum.reshape(NW, B, c), pcnt.reshape(NW, L), W.reshape(1, 3))

    nb = n // rb
    bidx3 = batch_idx.reshape(nb, 1, rb)
    out = pl.pallas_call(
        _p3,
        grid=(nb,),
        in_specs=[
            pl.BlockSpec((1, 1, rb), lambda i: (i, 0, 0)),
            pl.BlockSpec((rb, c), lambda i: (i, 0)),
            pl.BlockSpec((B, c), lambda i: (0, 0)),
        ],
        out_specs=pl.BlockSpec((rb, c), lambda i: (i, 0)),
        out_shape=jax.ShapeDtypeStruct((n, c), jnp.float32),
    )(bidx3, features, gates)
    return out


# speedup vs baseline: 2.6400x; 1.1173x over previous
import functools

import jax
import jax.numpy as jnp
from jax import lax
from jax.experimental import pallas as pl
from jax.experimental.pallas import tpu as pltpu
from jax.experimental.pallas import tpu_sc as plsc

B = 16
L = 16
NC, NS = 2, 16
NW = NC * NS


def _lane(vec, s, iota):
    return jnp.sum(jnp.where(iota == s, vec, 0))


def _seg_bounds(bidx_v, r, iota):
    lo = jnp.zeros((L,), jnp.int32)
    hi = jnp.full((L,), r, jnp.int32)
    steps = max(1, (r).bit_length())

    def body(_, carry):
        lo, hi = carry
        active = lo < hi
        mid = lax.div(lo + hi, 2)
        vals = plsc.load_gather(bidx_v, [jnp.minimum(mid, r - 1)])
        pred = vals < iota
        lo = jnp.where(active & pred, mid + 1, lo)
        hi = jnp.where(active & jnp.logical_not(pred), mid, hi)
        return lo, hi

    lo, hi = lax.fori_loop(0, steps, body, (lo, hi))
    starts = [_lane(lo, s, iota) for s in range(B)]
    ends = starts[1:] + [jnp.int32(r)]
    return starts, ends


def _make_sums_kernel(c, r, ch):
    nch = r // ch
    mesh = plsc.VectorSubcoreMesh(core_axis_name="c", subcore_axis_name="s")

    @functools.partial(
        pl.kernel,
        out_type=[
            jax.ShapeDtypeStruct((NW, B * c), jnp.float32),
            jax.ShapeDtypeStruct((NW * L,), jnp.float32),
        ],
        mesh=mesh,
        compiler_params=pltpu.CompilerParams(needs_layout_passes=False),
        scratch_types=[
            pltpu.VMEM((r,), jnp.int32),
            pltpu.VMEM((ch * c,), jnp.float32),
            pltpu.VMEM((ch * c,), jnp.float32),
            pltpu.VMEM((B * c,), jnp.float32),
            pltpu.VMEM((L,), jnp.float32),
            pltpu.SemaphoreType.DMA,
            pltpu.SemaphoreType.DMA,
            pltpu.SemaphoreType.DMA,
        ],
    )
    def sums_kernel(feat_hbm, bidx_hbm, psum_hbm, pcnt_hbm,
                    bidx_v, f0, f1, acc_v, cnt_v, semi, sem0, sem1):
        wid = lax.axis_index("c") * NS + lax.axis_index("s")
        base = wid * r
        iota = lax.iota(jnp.int32, L)
        zero16 = jnp.zeros((L,), jnp.float32)

        pltpu.async_copy(bidx_hbm.at[pl.ds(base, r)], bidx_v, semi)
        pltpu.async_copy(feat_hbm.at[pl.ds(base * c, ch * c)], f0, sem0)
        pltpu.async_copy(feat_hbm.at[pl.ds((base + ch) * c, ch * c)], f1, sem1)

        for q in range(B * c // L):
            acc_v[pl.ds(q * L, L)] = zero16

        pltpu.make_async_copy(bidx_hbm.at[pl.ds(base, r)], bidx_v, semi).wait()
        starts, ends = _seg_bounds(bidx_v, r, iota)

        cnt_f = zero16
        for s in range(B):
            cnt_f = jnp.where(iota == s,
                              (ends[s] - starts[s]).astype(jnp.float32), cnt_f)
        cnt_v[...] = cnt_f

        def chunk_pair(p, carry):
            for b in range(2):
                j = p * 2 + b
                cb = j * ch
                fb = f0 if b == 0 else f1
                semb = sem0 if b == 0 else sem1
                pltpu.make_async_copy(
                    feat_hbm.at[pl.ds((base + cb) * c, ch * c)], fb, semb
                ).wait()
                for s in range(B):
                    lo = jnp.maximum(starts[s] - cb, 0)
                    hi = jnp.minimum(ends[s] - cb, ch)

                    @pl.when(hi > lo)
                    def _(s=s, lo=lo, hi=hi, fb=fb):
                        acc8 = tuple(
                            acc_v[pl.ds(s * c + k * L, L)] for k in range(c // L)
                        )

                        @plsc.parallel_loop(lo, hi, unroll=2, carry=acc8)
                        def a8(rr, a8, fb=fb):
                            return tuple(
                                a + fb[pl.ds(rr * c + k * L, L)]
                                for k, a in enumerate(a8)
                            )

                        for k in range(c // L):
                            acc_v[pl.ds(s * c + k * L, L)] = a8[k]

                nxt = j + 2

                @pl.when(nxt < nch)
                def _(nxt=nxt, fb=fb, semb=semb):
                    pltpu.async_copy(
                        feat_hbm.at[pl.ds((base + nxt * ch) * c, ch * c)],
                        fb, semb)
            return carry

        lax.fori_loop(0, nch // 2, chunk_pair, 0)

        pltpu.sync_copy(acc_v, psum_hbm.at[wid])
        pltpu.sync_copy(cnt_v, pcnt_hbm.at[pl.ds(wid * L, L)])

    return sums_kernel


def _p1(bidx_ref, feat_ref, sums_ref, cnt_ref):
    i = pl.program_id(0)
    rb = feat_ref.shape[0]
    b = bidx_ref[0, 0, :]
    onehot = (b[:, None] == jax.lax.broadcasted_iota(jnp.int32, (rb, B), 1)
              ).astype(jnp.float32)
    part = jax.lax.dot_general(onehot, feat_ref[...],
                               (((0,), (0,)), ((), ())),
                               preferred_element_type=jnp.float32)
    pcnt = jnp.sum(onehot, axis=0)[:, None]

    @pl.when(i == 0)
    def _():
        sums_ref[...] = jnp.zeros_like(sums_ref)
        cnt_ref[...] = jnp.zeros_like(cnt_ref)

    sums_ref[...] += part
    cnt_ref[...] += jnp.broadcast_to(pcnt, cnt_ref.shape)


def _p3g(bidx_ref, feat_ref, tsums_ref, tcnt_ref, psum_ref, pcnt_ref, w_ref,
         out_ref, gates_ref):
    i = pl.program_id(0)
    rb = feat_ref.shape[0]

    @pl.when(i == 0)
    def _():
        sums = tsums_ref[...] + jnp.sum(psum_ref[...], axis=0)
        cnt = tcnt_ref[...] + jnp.sum(pcnt_ref[...], axis=0)[:, None]
        m = sums / jnp.maximum(cnt, 1.0)
        w0 = w_ref[0, 0]
        w1 = w_ref[0, 1]
        w2 = w_ref[0, 2]
        zero = jnp.zeros((m.shape[0], 1), jnp.float32)
        left = jnp.concatenate([zero, m[:, :-1]], axis=1)
        right = jnp.concatenate([m[:, 1:], zero], axis=1)
        y = w0 * left + w1 * m + w2 * right
        gates_ref[...] = jax.nn.sigmoid(y)

    b = bidx_ref[0, 0, :]
    onehot = (b[:, None] == jax.lax.broadcasted_iota(jnp.int32, (rb, B), 1)
              ).astype(jnp.float32)
    g = jax.lax.dot_general(onehot, gates_ref[...],
                            (((1,), (0,)), ((), ())),
                            preferred_element_type=jnp.float32)
    out_ref[...] = feat_ref[...] * g


N_SC_FRAC_NUM, N_SC_FRAC_DEN = 1, 2


def kernel(features, batch_idx, W):
    n, c = features.shape
    assert c % L == 0
    rb = 4000
    n_sc = (n * N_SC_FRAC_NUM // N_SC_FRAC_DEN) // (NW * rb) * (NW * rb) \
        // rb * rb
    n_sc = max(n_sc, NW * 500)
    r = n_sc // NW
    ch_a = 250
    assert r % ch_a == 0 and (r // ch_a) % 2 == 0, (r, ch_a)
    n_tc = n - n_sc

    featflat = features.reshape(-1)

    psum, pcnt = _make_sums_kernel(c, r, ch_a)(featflat, batch_idx)

    assert n_tc % rb == 0 and n_sc % rb == 0
    nb_tc = n_tc // rb
    blk0 = n_sc // rb
    bidx3_tc = batch_idx[n_sc:].reshape(nb_tc, 1, rb)
    tsums, tcnt = pl.pallas_call(
        _p1,
        grid=(nb_tc,),
        in_specs=[
            pl.BlockSpec((1, 1, rb), lambda i: (i, 0, 0)),
            pl.BlockSpec((rb, c), lambda i: (i + blk0, 0)),
        ],
        out_specs=[
            pl.BlockSpec((B, c), lambda i: (0, 0)),
            pl.BlockSpec((B, c), lambda i: (0, 0)),
        ],
        out_shape=[
            jax.ShapeDtypeStruct((B, c), jnp.float32),
            jax.ShapeDtypeStruct((B, c), jnp.float32),
        ],
    )(bidx3_tc, features)

    rb3 = 8000
    assert n % rb3 == 0
    nb = n // rb3
    bidx3 = batch_idx.reshape(nb, 1, rb3)
    out = pl.pallas_call(
        _p3g,
        grid=(nb,),
        in_specs=[
            pl.BlockSpec((1, 1, rb3), lambda i: (i, 0, 0)),
            pl.BlockSpec((rb3, c), lambda i: (i, 0)),
            pl.BlockSpec((B, c), lambda i: (0, 0)),
            pl.BlockSpec((B, c), lambda i: (0, 0)),
            pl.BlockSpec((NW, B, c), lambda i: (0, 0, 0)),
            pl.BlockSpec((NW, L), lambda i: (0, 0)),
            pl.BlockSpec((1, 3), lambda i: (0, 0)),
        ],
        out_specs=pl.BlockSpec((rb3, c), lambda i: (i, 0)),
        out_shape=jax.ShapeDtypeStruct((n, c), jnp.float32),
        scratch_shapes=[pltpu.VMEM((B, c), jnp.float32)],
    )(bidx3, features, tsums, tcnt, psum.reshape(NW, B, c),
      pcnt.reshape(NW, L), W.reshape(1, 3))
    return out


# speedup vs baseline: 2.8096x; 1.0643x over previous
import functools

import jax
import jax.numpy as jnp
from jax import lax
from jax.experimental import pallas as pl
from jax.experimental.pallas import tpu as pltpu
from jax.experimental.pallas import tpu_sc as plsc

B = 16
L = 16
NC, NS = 2, 16
NW = NC * NS


def _lane(vec, s, iota):
    return jnp.sum(jnp.where(iota == s, vec, 0))


def _seg_bounds(bidx_v, r, iota):
    lo = jnp.zeros((L,), jnp.int32)
    hi = jnp.full((L,), r, jnp.int32)
    steps = max(1, (r).bit_length())

    def body(_, carry):
        lo, hi = carry
        active = lo < hi
        mid = lax.div(lo + hi, 2)
        vals = plsc.load_gather(bidx_v, [jnp.minimum(mid, r - 1)])
        pred = vals < iota
        lo = jnp.where(active & pred, mid + 1, lo)
        hi = jnp.where(active & jnp.logical_not(pred), mid, hi)
        return lo, hi

    lo, hi = lax.fori_loop(0, steps, body, (lo, hi))
    starts = [_lane(lo, s, iota) for s in range(B)]
    ends = starts[1:] + [jnp.int32(r)]
    return starts, ends


def _make_sums_kernel(c, r, ch):
    nch = r // ch
    mesh = plsc.VectorSubcoreMesh(core_axis_name="c", subcore_axis_name="s")

    @functools.partial(
        pl.kernel,
        out_type=[
            jax.ShapeDtypeStruct((NW, B * c), jnp.float32),
            jax.ShapeDtypeStruct((NW * L,), jnp.float32),
        ],
        mesh=mesh,
        compiler_params=pltpu.CompilerParams(needs_layout_passes=False),
        scratch_types=[
            pltpu.VMEM((r,), jnp.int32),
            pltpu.VMEM((ch * c,), jnp.float32),
            pltpu.VMEM((ch * c,), jnp.float32),
            pltpu.VMEM((B * c,), jnp.float32),
            pltpu.VMEM((L,), jnp.float32),
            pltpu.SemaphoreType.DMA,
            pltpu.SemaphoreType.DMA,
            pltpu.SemaphoreType.DMA,
        ],
    )
    def sums_kernel(feat_hbm, bidx_hbm, psum_hbm, pcnt_hbm,
                    bidx_v, f0, f1, acc_v, cnt_v, semi, sem0, sem1):
        wid = lax.axis_index("c") * NS + lax.axis_index("s")
        base = wid * r
        iota = lax.iota(jnp.int32, L)
        zero16 = jnp.zeros((L,), jnp.float32)

        pltpu.async_copy(bidx_hbm.at[pl.ds(base, r)], bidx_v, semi)
        pltpu.async_copy(feat_hbm.at[pl.ds(base * c, ch * c)], f0, sem0)
        pltpu.async_copy(feat_hbm.at[pl.ds((base + ch) * c, ch * c)], f1, sem1)

        for q in range(B * c // L):
            acc_v[pl.ds(q * L, L)] = zero16

        pltpu.make_async_copy(bidx_hbm.at[pl.ds(base, r)], bidx_v, semi).wait()
        starts, ends = _seg_bounds(bidx_v, r, iota)

        cnt_f = zero16
        for s in range(B):
            cnt_f = jnp.where(iota == s,
                              (ends[s] - starts[s]).astype(jnp.float32), cnt_f)
        cnt_v[...] = cnt_f

        def chunk_pair(p, carry):
            for b in range(2):
                j = p * 2 + b
                cb = j * ch
                fb = f0 if b == 0 else f1
                semb = sem0 if b == 0 else sem1
                pltpu.make_async_copy(
                    feat_hbm.at[pl.ds((base + cb) * c, ch * c)], fb, semb
                ).wait()
                for s in range(B):
                    lo = jnp.maximum(starts[s] - cb, 0)
                    hi = jnp.minimum(ends[s] - cb, ch)

                    @pl.when(hi > lo)
                    def _(s=s, lo=lo, hi=hi, fb=fb):
                        acc8 = tuple(
                            acc_v[pl.ds(s * c + k * L, L)] for k in range(c // L)
                        )

                        @plsc.parallel_loop(lo, hi, unroll=2, carry=acc8)
                        def a8(rr, a8, fb=fb):
                            return tuple(
                                a + fb[pl.ds(rr * c + k * L, L)]
                                for k, a in enumerate(a8)
                            )

                        for k in range(c // L):
                            acc_v[pl.ds(s * c + k * L, L)] = a8[k]

                nxt = j + 2

                @pl.when(nxt < nch)
                def _(nxt=nxt, fb=fb, semb=semb):
                    pltpu.async_copy(
                        feat_hbm.at[pl.ds((base + nxt * ch) * c, ch * c)],
                        fb, semb)
            return carry

        lax.fori_loop(0, nch // 2, chunk_pair, 0)

        pltpu.sync_copy(acc_v, psum_hbm.at[wid])
        pltpu.sync_copy(cnt_v, pcnt_hbm.at[pl.ds(wid * L, L)])

    return sums_kernel


def _p1(bidx_ref, feat_ref, sums_ref, cnt_ref):
    i = pl.program_id(0)
    rb = feat_ref.shape[0]
    b = bidx_ref[0, 0, :]
    onehot = (b[:, None] == jax.lax.broadcasted_iota(jnp.int32, (rb, B), 1)
              ).astype(jnp.float32)
    part = jax.lax.dot_general(onehot, feat_ref[...],
                               (((0,), (0,)), ((), ())),
                               preferred_element_type=jnp.float32)
    pcnt = jnp.sum(onehot, axis=0)[:, None]

    @pl.when(i == 0)
    def _():
        sums_ref[...] = jnp.zeros_like(sums_ref)
        cnt_ref[...] = jnp.zeros_like(cnt_ref)

    sums_ref[...] += part
    cnt_ref[...] += jnp.broadcast_to(pcnt, cnt_ref.shape)


def _p3g(bidx_ref, feat_ref, tsums_ref, tcnt_ref, psum_ref, pcnt_ref, w_ref,
         out_ref, gates_ref):
    i = pl.program_id(0)
    rb = feat_ref.shape[0]

    @pl.when(i == 0)
    def _():
        sums = tsums_ref[...] + jnp.sum(psum_ref[...], axis=0)
        cnt = tcnt_ref[...] + jnp.sum(pcnt_ref[...], axis=0)[:, None]
        m = sums / jnp.maximum(cnt, 1.0)
        w0 = w_ref[0, 0]
        w1 = w_ref[0, 1]
        w2 = w_ref[0, 2]
        zero = jnp.zeros((m.shape[0], 1), jnp.float32)
        left = jnp.concatenate([zero, m[:, :-1]], axis=1)
        right = jnp.concatenate([m[:, 1:], zero], axis=1)
        y = w0 * left + w1 * m + w2 * right
        gates_ref[...] = jax.nn.sigmoid(y)

    b = bidx_ref[0, 0, :]
    onehot = (b[:, None] == jax.lax.broadcasted_iota(jnp.int32, (rb, B), 1)
              ).astype(jnp.float32)
    g = jax.lax.dot_general(onehot, gates_ref[...],
                            (((1,), (0,)), ((), ())),
                            preferred_element_type=jnp.float32)
    out_ref[...] = feat_ref[...] * g


N_SC_FRAC_NUM, N_SC_FRAC_DEN = 1, 2


def kernel(features, batch_idx, W):
    n, c = features.shape
    assert c % L == 0
    rb = 8000
    grain = NW * 1000
    n_sc = max(grain, (n * N_SC_FRAC_NUM // N_SC_FRAC_DEN) // grain * grain)
    r = n_sc // NW
    ch_a = 250
    assert r % ch_a == 0 and (r // ch_a) % 2 == 0 and r % 8 == 0, (r, ch_a)
    n_tc = n - n_sc

    featflat = features.reshape(-1)

    psum, pcnt = _make_sums_kernel(c, r, ch_a)(featflat, batch_idx)

    assert n_tc % rb == 0 and n_sc % rb == 0
    nb_tc = n_tc // rb
    blk0 = n_sc // rb
    bidx3_tc = batch_idx[n_sc:].reshape(nb_tc, 1, rb)
    tsums, tcnt = pl.pallas_call(
        _p1,
        grid=(nb_tc,),
        in_specs=[
            pl.BlockSpec((1, 1, rb), lambda i: (i, 0, 0)),
            pl.BlockSpec((rb, c), lambda i: (i + blk0, 0)),
        ],
        out_specs=[
            pl.BlockSpec((B, c), lambda i: (0, 0)),
            pl.BlockSpec((B, c), lambda i: (0, 0)),
        ],
        out_shape=[
            jax.ShapeDtypeStruct((B, c), jnp.float32),
            jax.ShapeDtypeStruct((B, c), jnp.float32),
        ],
    )(bidx3_tc, features)

    rb3 = 16000
    assert n % rb3 == 0
    nb = n // rb3
    bidx3 = batch_idx.reshape(nb, 1, rb3)
    out = pl.pallas_call(
        _p3g,
        grid=(nb,),
        in_specs=[
            pl.BlockSpec((1, 1, rb3), lambda i: (i, 0, 0)),
            pl.BlockSpec((rb3, c), lambda i: (i, 0)),
            pl.BlockSpec((B, c), lambda i: (0, 0)),
            pl.BlockSpec((B, c), lambda i: (0, 0)),
            pl.BlockSpec((NW, B, c), lambda i: (0, 0, 0)),
            pl.BlockSpec((NW, L), lambda i: (0, 0)),
            pl.BlockSpec((1, 3), lambda i: (0, 0)),
        ],
        out_specs=pl.BlockSpec((rb3, c), lambda i: (i, 0)),
        out_shape=jax.ShapeDtypeStruct((n, c), jnp.float32),
        scratch_shapes=[pltpu.VMEM((B, c), jnp.float32)],
    )(bidx3, features, tsums, tcnt, psum.reshape(NW, B, c),
      pcnt.reshape(NW, L), W.reshape(1, 3))
    return out
